# baseline (speedup 1.0000x reference)
import functools

import jax
import jax.numpy as jnp
from jax import lax
from jax.experimental import pallas as pl
from jax.experimental.pallas import tpu as pltpu
from jax.experimental.pallas import tpu_sc as plsc

B = 128
HB = B // 2
N = 100000
KTH = 10
L = 16
NW = 32
RW = HB // NW
CHW = 12672
LASTW = 11264
TAILW = 32
LASTF = LASTW + 128
NCH = 8
GE = 1408
GV = GE // L
NGM = CHW // GE
NGL = LASTW // GE
SUB = 11
NSUB = GV // SUB
KCH = RW * NCH
CAND = 4096
NEG = float("-inf")

_mesh = plsc.VectorSubcoreMesh(core_axis_name="c", subcore_axis_name="s")
_cparams = pltpu.CompilerParams(needs_layout_passes=False,
                                disable_bounds_checks=True)


def _merge_top16(best_asc, vec):
    v_desc = lax.rev(lax.sort(vec), (0,))
    return lax.sort(jnp.maximum(best_asc, v_desc))


def _any_above(vec, thr):
    return plsc.all_reduce_population_count(vec > thr)[0] > 0


def _make_topk(base_row):
    @functools.partial(
        pl.kernel,
        out_type=[
            jax.ShapeDtypeStruct((NW, L), jnp.float32),
            jax.ShapeDtypeStruct((NW, L), jnp.float32),
        ],
        mesh=_mesh,
        compiler_params=_cparams,
        scratch_types=[
            pltpu.VMEM((2 * CHW,), jnp.float32),
            pltpu.VMEM((NGM * L,), jnp.float32),
            pltpu.VMEM((NGM * NSUB * L,), jnp.float32),
            pltpu.VMEM((B,), jnp.int32),
            pltpu.VMEM((CAND,), jnp.float32),
            pltpu.VMEM((L,), jnp.float32),
            pltpu.VMEM((L,), jnp.float32),
            pltpu.SemaphoreType.DMA,
            pltpu.SemaphoreType.DMA,
        ],
    )
    def _topk_gather(x_hbm, y_hbm, stopk_hbm, sy_hbm,
                     buf, gsum_v, ssum_v, y_v, cand_v, tk_v, sy_v,
                     sem0, sem1):
        wid = lax.axis_index("s") * 2 + lax.axis_index("c")
        row0 = wid * RW
        pltpu.sync_copy(y_hbm, y_v)
        iota = lax.iota(jnp.int32, L)
        sems = (sem0, sem1)

        def xrow(k):
            return x_hbm.at[row0 + k // NCH]

        def src_main(k):
            return xrow(k).at[pl.ds(pl.multiple_of((k % NCH) * CHW, 128),
                                    CHW)]

        def src_last(k):
            col = pl.multiple_of((NCH - 1) * CHW + 0 * k, 128)
            return xrow(k).at[pl.ds(col, LASTF)]

        def dst_main(h):
            return buf.at[pl.ds(h * CHW, CHW)]

        def dst_last(h):
            return buf.at[pl.ds(h * CHW, LASTF)]

        def issue(k, h):
            c2 = k % NCH

            @pl.when(jnp.logical_and(k < KCH, c2 < NCH - 1))
            def _():
                pltpu.async_copy(src_main(k), dst_main(h), sems[h])

            @pl.when(jnp.logical_and(k < KCH, c2 == NCH - 1))
            def _():
                pltpu.async_copy(src_last(k), dst_last(h), sems[h])

        def wait(k, h):
            c2 = k % NCH

            @pl.when(c2 < NCH - 1)
            def _():
                pltpu.make_async_copy(src_main(k), dst_main(h),
                                      sems[h]).wait()

            @pl.when(c2 == NCH - 1)
            def _():
                pltpu.make_async_copy(src_last(k), dst_last(h),
                                      sems[h]).wait()

        issue(0, 0)
        issue(1, 1)

        def body(k, carry):
            bestc, coff, syv, tk_res, sy_res = carry
            par = k % 2
            dbase = par * CHW
            c = k % NCH
            r_loc = k // NCH
            last = c == NCH - 1
            gend = jnp.where(last, NGL, NGM)

            @pl.when(par == 0)
            def _():
                wait(k, 0)

            @pl.when(par == 1)
            def _():
                wait(k, 1)

            def g_body(gi, best):
                base = dbase + gi * GE
                subs = []
                for sg in range(NSUB):
                    sb = base + sg * (SUB * L)
                    a0 = buf[pl.ds(sb, L)]
                    a1 = buf[pl.ds(sb + L, L)]
                    for j in range(2, SUB):
                        if j % 2 == 0:
                            a0 = jnp.maximum(a0, buf[pl.ds(sb + j * L, L)])
                        else:
                            a1 = jnp.maximum(a1, buf[pl.ds(sb + j * L, L)])
                    ms = jnp.maximum(a0, a1)
                    ssum_v[pl.ds((gi * NSUB + sg) * L, L)] = ms
                    subs.append(ms)
                m0 = jnp.maximum(jnp.maximum(subs[0], subs[1]),
                                 jnp.maximum(subs[2], subs[3]))
                m1 = jnp.maximum(jnp.maximum(subs[4], subs[5]),
                                 jnp.maximum(subs[6], subs[7]))
                m = jnp.maximum(m0, m1)
                gsum_v[pl.ds(gi * L, L)] = m
                return lax.cond(_any_above(m, best[0]),
                                lambda b: _merge_top16(b, m),
                                lambda b: b, best)

            bestc = lax.fori_loop(0, gend, g_body, bestc)
            u = bestc[L - 1 - KTH]

            def d_body(gi, off):
                gm = gsum_v[pl.ds(gi * L, L)]

                def dig(off1):
                    for sg in range(NSUB):
                        sm = ssum_v[pl.ds((gi * NSUB + sg) * L, L)]

                        def dig2(off2):
                            sb = dbase + gi * GE + sg * (SUB * L)

                            def v_body(j, off3):
                                v = buf[pl.ds(sb + j * L, L)]
                                msk = v > u
                                plsc.store_compressed(
                                    cand_v.at[pl.ds(off3, L)], v, mask=msk)
                                return off3 + \
                                    plsc.all_reduce_population_count(msk)[0]

                            return lax.fori_loop(0, SUB, v_body, off2)

                        off1 = lax.cond(_any_above(sm, u), dig2,
                                        lambda o: o, off1)
                    return off1

                return lax.cond(_any_above(gm, u), dig, lambda o: o, off)

            gmask = gsum_v[pl.ds(0, L)] > u

            def or_body(gi, msk):
                return jnp.logical_or(msk, gsum_v[pl.ds(gi * L, L)] > u)

            gmask = lax.fori_loop(1, gend, or_body, gmask)
            coff = lax.cond(
                plsc.all_reduce_population_count(gmask)[0] > 0,
                lambda o: lax.fori_loop(0, gend, d_body, o),
                lambda o: o, coff)

            row = base_row + row0 + r_loc
            yvec = y_v[pl.ds((row // L) * L, L)]
            yi = jnp.max(jnp.where(iota == row % L, yvec, jnp.int32(-1)))
            q = yi - c * CHW
            climit = jnp.where(last, LASTW + TAILW, CHW)
            valid = jnp.logical_and(q >= 0, q < climit)
            qc = jnp.maximum(jnp.minimum(q, CHW - 1), 0)
            vsel = buf[pl.ds(dbase + (qc // L) * L, L)]
            sel = jnp.max(jnp.where(iota == qc % L, vsel, NEG))
            syv = jnp.where(valid, sel, syv)

            @pl.when(par == 0)
            def _():
                issue(k + 2, 0)

            @pl.when(par == 1)
            def _():
                issue(k + 2, 1)

            def finalize(args):
                bestc, coff, syv, tk_res, sy_res = args

                def m_body(i, mg):
                    cv = cand_v[pl.ds(i * L, L)]
                    cv = jnp.where(iota < coff - i * L, cv, NEG)
                    return _merge_top16(mg, cv)

                nvec = (coff + L - 1) // L
                merged = lax.fori_loop(0, nvec, m_body,
                                       jnp.full((L,), NEG, jnp.float32))
                tv0 = buf[pl.ds(dbase + LASTW, L)]
                tv1 = buf[pl.ds(dbase + LASTW + L, L)]
                merged = _merge_top16(_merge_top16(merged, tv0), tv1)
                t_max = bestc[L - 1 - KTH]
                cnt = plsc.all_reduce_population_count(merged > t_max)[0]
                ans = jnp.where(cnt <= KTH, t_max, merged[L - 1 - KTH])
                done = iota == r_loc
                tk_res = jnp.where(done, ans, tk_res)
                sy_res = jnp.where(done, syv, sy_res)
                return (jnp.full((L,), NEG, jnp.float32), jnp.int32(0),
                        jnp.float32(0), tk_res, sy_res)

            bestc, coff, syv, tk_res, sy_res = lax.cond(
                last, finalize, lambda a: a,
                (bestc, coff, syv, tk_res, sy_res))
            return (bestc, coff, syv, tk_res, sy_res)

        init = (jnp.full((L,), NEG, jnp.float32), jnp.int32(0),
                jnp.float32(0), jnp.full((L,), NEG, jnp.float32),
                jnp.full((L,), NEG, jnp.float32))
        _, _, _, tk_res, sy_res = lax.fori_loop(0, KCH, body, init)

        tk_v[...] = tk_res
        sy_v[...] = sy_res
        pltpu.sync_copy(tk_v, stopk_hbm.at[wid])
        pltpu.sync_copy(sy_v, sy_hbm.at[wid])

    return _topk_gather


_topk_lo = _make_topk(0)
_topk_hi = _make_topk(HB)


@functools.partial(
    pl.kernel,
    out_type=jax.ShapeDtypeStruct((L,), jnp.float32),
    mesh=_mesh,
    compiler_params=_cparams,
    scratch_types=[
        pltpu.VMEM((NW, L), jnp.float32),
        pltpu.VMEM((NW, L), jnp.float32),
        pltpu.VMEM((NW, L), jnp.float32),
        pltpu.VMEM((NW, L), jnp.float32),
        pltpu.VMEM((B,), jnp.float32),
        pltpu.VMEM((L,), jnp.float32),
    ],
)
def _pair_mean(tka_hbm, tkb_hbm, sya_hbm, syb_hbm, out_hbm,
               tka_v, tkb_v, sya_v, syb_v, syc_v, o_v):
    wid = lax.axis_index("s") * 2 + lax.axis_index("c")

    @pl.when(wid == 0)
    def _():
        pltpu.sync_copy(tka_hbm, tka_v)
        pltpu.sync_copy(tkb_hbm, tkb_v)
        pltpu.sync_copy(sya_hbm, sya_v)
        pltpu.sync_copy(syb_hbm, syb_v)
        iota = lax.iota(jnp.int32, L)
        tvs = []
        for j in range(B // L):
            g = j * L + iota
            gr = (g % HB) // RW
            gc = g % RW
            srcs = (tka_v, sya_v) if j < (HB // L) else (tkb_v, syb_v)
            tvs.append(1.0 + plsc.load_gather(srcs[0], (gr, gc)))
            syc_v[pl.ds(j * L, L)] = plsc.load_gather(srcs[1], (gr, gc))

        def i_body(i, acc):
            svec = syc_v[pl.ds((i // L) * L, L)]
            syi = jnp.max(jnp.where(iota == i % L, svec, NEG))
            for j in range(B // L):
                acc = acc + jnp.maximum(tvs[j] - syi, 0.0)
            return acc

        acc = lax.fori_loop(0, B, i_body, jnp.zeros((L,), jnp.float32))
        total = jnp.sum(acc)
        o_v[...] = jnp.full((L,), total * (1.0 / (B * B)), jnp.float32)
        pltpu.sync_copy(o_v, out_hbm)


def kernel(x, y):
    y32 = y.astype(jnp.int32)
    tka, sya = _topk_lo(x[:HB], y32)
    tkb, syb = _topk_hi(x[HB:], y32)
    out = _pair_mean(tka, tkb, sya, syb)
    return out[0]


# speedup vs baseline: 1.5242x; 1.5242x over previous
import functools

import jax
import jax.numpy as jnp
from jax import lax
from jax.experimental import pallas as pl
from jax.experimental.pallas import tpu as pltpu
from jax.experimental.pallas import tpu_sc as plsc

B = 128
N = 100000
KTH = 10
L = 16
NW = 32
ROWS_PER_W = B // NW
CHW = 12672
LASTW = 11264
TAILW = 32
NCH = 8
GE = 1408
GV = GE // L
NGM = CHW // GE
NGL = LASTW // GE
SUB = 11
NSUB = GV // SUB
KCHUNKS = ROWS_PER_W * NCH
NEG = float("-inf")

_mesh = plsc.VectorSubcoreMesh(core_axis_name="c", subcore_axis_name="s")
_cparams = pltpu.CompilerParams(needs_layout_passes=False,
                                disable_bounds_checks=True)
LASTF = LASTW + 128
CAND = 8192


def _merge_top16(best_asc, vec):
    v_desc = lax.rev(lax.sort(vec), (0,))
    return lax.sort(jnp.maximum(best_asc, v_desc))


def _any_above(vec, thr):
    return plsc.all_reduce_population_count(vec > thr)[0] > 0


@functools.partial(
    pl.kernel,
    out_type=[
        jax.ShapeDtypeStruct((NW, L), jnp.float32),
        jax.ShapeDtypeStruct((NW, L), jnp.float32),
    ],
    mesh=_mesh,
    compiler_params=_cparams,
    scratch_types=[
        pltpu.VMEM((2 * CHW, ), jnp.float32),
        pltpu.VMEM((NGM * L,), jnp.float32),
        pltpu.VMEM((NGM * NSUB * L,), jnp.float32),
        pltpu.VMEM((B,), jnp.int32),
        pltpu.VMEM((CAND,), jnp.float32),
        pltpu.VMEM((L,), jnp.float32),
        pltpu.VMEM((L,), jnp.float32),
        pltpu.SemaphoreType.DMA,
        pltpu.SemaphoreType.DMA,
    ],
)
def _topk_gather(x_hbm, y_hbm, stopk_hbm, sy_hbm,
                 buf, gsum_v, ssum_v, y_v, cand_v, tk_v, sy_v, sem0, sem1):
    wid = lax.axis_index("s") * 2 + lax.axis_index("c")
    row0 = wid * ROWS_PER_W
    pltpu.sync_copy(y_hbm, y_v)
    iota = lax.iota(jnp.int32, L)
    sems = (sem0, sem1)

    def xrow(k):
        return x_hbm.at[row0 + k // NCH]

    def src_main(k):
        return xrow(k).at[pl.ds(pl.multiple_of((k % NCH) * CHW, 128), CHW)]

    def src_last(k):
        col = pl.multiple_of((NCH - 1) * CHW + 0 * k, 128)
        return xrow(k).at[pl.ds(col, LASTF)]

    def dst_main(h):
        return buf.at[pl.ds(h * CHW, CHW)]

    def dst_last(h):
        return buf.at[pl.ds(h * CHW, LASTF)]

    def issue(k, h):
        c2 = k % NCH

        @pl.when(jnp.logical_and(k < KCHUNKS, c2 < NCH - 1))
        def _():
            pltpu.async_copy(src_main(k), dst_main(h), sems[h])

        @pl.when(jnp.logical_and(k < KCHUNKS, c2 == NCH - 1))
        def _():
            pltpu.async_copy(src_last(k), dst_last(h), sems[h])

    def wait(k, h):
        c2 = k % NCH

        @pl.when(c2 < NCH - 1)
        def _():
            pltpu.make_async_copy(src_main(k), dst_main(h), sems[h]).wait()

        @pl.when(c2 == NCH - 1)
        def _():
            pltpu.make_async_copy(src_last(k), dst_last(h), sems[h]).wait()

    issue(0, 0)
    issue(1, 1)

    def body(k, carry):
        bestc, coff, syv, tk_res, sy_res = carry
        par = k % 2
        dbase = par * CHW
        c = k % NCH
        r_loc = k // NCH
        last = c == NCH - 1
        gend = jnp.where(last, NGL, NGM)

        @pl.when(par == 0)
        def _():
            wait(k, 0)

        @pl.when(par == 1)
        def _():
            wait(k, 1)

        def g_body(gi, best):
            base = dbase + gi * GE
            subs = []
            for sg in range(NSUB):
                sb = base + sg * (SUB * L)
                a0 = buf[pl.ds(sb, L)]
                a1 = buf[pl.ds(sb + L, L)]
                for j in range(2, SUB):
                    if j % 2 == 0:
                        a0 = jnp.maximum(a0, buf[pl.ds(sb + j * L, L)])
                    else:
                        a1 = jnp.maximum(a1, buf[pl.ds(sb + j * L, L)])
                ms = jnp.maximum(a0, a1)
                ssum_v[pl.ds((gi * NSUB + sg) * L, L)] = ms
                subs.append(ms)
            m0 = jnp.maximum(jnp.maximum(subs[0], subs[1]),
                             jnp.maximum(subs[2], subs[3]))
            m1 = jnp.maximum(jnp.maximum(subs[4], subs[5]),
                             jnp.maximum(subs[6], subs[7]))
            m = jnp.maximum(m0, m1)
            gsum_v[pl.ds(gi * L, L)] = m
            return lax.cond(_any_above(m, best[0]),
                            lambda b: _merge_top16(b, m), lambda b: b, best)

        bestc = lax.fori_loop(0, gend, g_body, bestc)
        u = bestc[L - 1 - KTH]

        def d_body(gi, off):
            gm = gsum_v[pl.ds(gi * L, L)]

            def dig(off1):
                sms = [ssum_v[pl.ds((gi * NSUB + sg) * L, L)]
                       for sg in range(NSUB)]
                hits = [sm > u for sm in sms]

                def collect(sg):
                    def dig2(off2):
                        sb = dbase + gi * GE + sg * (SUB * L)

                        def v_body(j, off3):
                            v = buf[pl.ds(sb + j * L, L)]
                            msk = v > u
                            plsc.store_compressed(
                                cand_v.at[pl.ds(off3, L)], v, mask=msk)
                            return off3 + plsc.all_reduce_population_count(
                                msk)[0]

                        return lax.fori_loop(0, SUB, v_body, off2)

                    return dig2

                for bq in range(2):
                    h4 = jnp.logical_or(
                        jnp.logical_or(hits[4 * bq], hits[4 * bq + 1]),
                        jnp.logical_or(hits[4 * bq + 2], hits[4 * bq + 3]))

                    def batch(off2, bq=bq):
                        for sg in range(4 * bq, 4 * bq + 4):
                            off2 = lax.cond(
                                plsc.all_reduce_population_count(
                                    hits[sg])[0] > 0,
                                collect(sg), lambda o: o, off2)
                        return off2

                    off1 = lax.cond(
                        plsc.all_reduce_population_count(h4)[0] > 0,
                        batch, lambda o: o, off1)
                return off1

            return lax.cond(_any_above(gm, u), dig, lambda o: o, off)

        gmask = gsum_v[pl.ds(0, L)] > u

        def or_body(gi, msk):
            return jnp.logical_or(msk, gsum_v[pl.ds(gi * L, L)] > u)

        gmask = lax.fori_loop(1, gend, or_body, gmask)
        coff = lax.cond(
            plsc.all_reduce_population_count(gmask)[0] > 0,
            lambda o: lax.fori_loop(0, gend, d_body, o),
            lambda o: o, coff)

        row = row0 + r_loc
        yvec = y_v[pl.ds((row // L) * L, L)]
        yi = jnp.max(jnp.where(iota == row % L, yvec, jnp.int32(-1)))
        q = yi - c * CHW
        climit = jnp.where(last, LASTW + TAILW, CHW)
        valid = jnp.logical_and(q >= 0, q < climit)
        qc = jnp.maximum(jnp.minimum(q, CHW - 1), 0)
        vsel = buf[pl.ds(dbase + (qc // L) * L, L)]
        sel = jnp.max(jnp.where(iota == qc % L, vsel, NEG))
        syv = jnp.where(valid, sel, syv)

        @pl.when(par == 0)
        def _():
            issue(k + 2, 0)

        @pl.when(par == 1)
        def _():
            issue(k + 2, 1)

        def finalize(args):
            bestc, coff, syv, tk_res, sy_res = args

            def m_body(i, mg):
                cv = cand_v[pl.ds(i * L, L)]
                cv = jnp.where(iota < coff - i * L, cv, NEG)
                return _merge_top16(mg, cv)

            nvec = (coff + L - 1) // L
            merged = lax.fori_loop(0, nvec, m_body,
                                   jnp.full((L,), NEG, jnp.float32))
            tv0 = buf[pl.ds(dbase + LASTW, L)]
            tv1 = buf[pl.ds(dbase + LASTW + L, L)]
            merged = _merge_top16(_merge_top16(merged, tv0), tv1)
            t_max = bestc[L - 1 - KTH]
            cnt = plsc.all_reduce_population_count(merged > t_max)[0]
            ans = jnp.where(cnt <= KTH, t_max, merged[L - 1 - KTH])
            done = iota == r_loc
            tk_res = jnp.where(done, ans, tk_res)
            sy_res = jnp.where(done, syv, sy_res)
            return (jnp.full((L,), NEG, jnp.float32), jnp.int32(0),
                    jnp.float32(0), tk_res, sy_res)

        bestc, coff, syv, tk_res, sy_res = lax.cond(
            last, finalize, lambda a: a,
            (bestc, coff, syv, tk_res, sy_res))
        return (bestc, coff, syv, tk_res, sy_res)

    init = (jnp.full((L,), NEG, jnp.float32), jnp.int32(0),
            jnp.float32(0), jnp.full((L,), NEG, jnp.float32),
            jnp.full((L,), NEG, jnp.float32))
    _, _, _, tk_res, sy_res = lax.fori_loop(0, KCHUNKS, body, init)

    tk_v[...] = tk_res
    sy_v[...] = sy_res
    pltpu.sync_copy(tk_v, stopk_hbm.at[wid])
    pltpu.sync_copy(sy_v, sy_hbm.at[wid])


@functools.partial(
    pl.kernel,
    out_type=jax.ShapeDtypeStruct((L,), jnp.float32),
    mesh=_mesh,
    compiler_params=_cparams,
    scratch_types=[
        pltpu.VMEM((NW, L), jnp.float32),
        pltpu.VMEM((NW, L), jnp.float32),
        pltpu.VMEM((L,), jnp.float32),
    ],
)
def _pair_mean(stopk_hbm, sy_hbm, out_hbm, tk_v, sy_v, o_v):
    wid = lax.axis_index("s") * 2 + lax.axis_index("c")

    @pl.when(wid == 0)
    def _():
        pltpu.sync_copy(stopk_hbm, tk_v)
        pltpu.sync_copy(sy_hbm, sy_v)
        tvs = [1.0 + tk_v[w] for w in range(NW)]
        iota = lax.iota(jnp.int32, L)

        def i_body(i, acc):
            svec = sy_v[i // ROWS_PER_W]
            syi = jnp.max(jnp.where(iota == i % ROWS_PER_W, svec, NEG))
            for w in range(NW):
                acc = acc + jnp.maximum(tvs[w] - syi, 0.0)
            return acc

        acc = lax.fori_loop(0, B, i_body, jnp.zeros((L,), jnp.float32))
        total = jnp.sum(acc)
        o_v[...] = jnp.full((L,), total * (1.0 / (B * B)), jnp.float32)
        pltpu.sync_copy(o_v, out_hbm)


def kernel(x, y):
    stopk, sy = _topk_gather(x, y.astype(jnp.int32))
    out = _pair_mean(stopk, sy)
    return out[0]
